# baseline (speedup 1.0000x reference)
import jax
import jax.numpy as jnp
from jax.experimental import pallas as pl

NUM_VIEWS = 12
PROJECTION_DIM = 512
BATCH = 4096
FLAT = NUM_VIEWS * PROJECTION_DIM

BLOCK_B = 256


def _body(x_ref, p_ref, o_ref):
    o_ref[...] = x_ref[...] + p_ref[...]


def kernel(onedimage, pos_table):
    x = onedimage.reshape(BATCH, FLAT)
    p = pos_table.reshape(1, FLAT)
    out = pl.pallas_call(
        _body,
        grid=(BATCH // BLOCK_B,),
        in_specs=[
            pl.BlockSpec((BLOCK_B, FLAT), lambda i: (i, 0)),
            pl.BlockSpec((1, FLAT), lambda i: (0, 0)),
        ],
        out_specs=pl.BlockSpec((BLOCK_B, FLAT), lambda i: (i, 0)),
        out_shape=jax.ShapeDtypeStruct((BATCH, FLAT), jnp.float32),
    )(x, p)
    return out.reshape(BATCH, NUM_VIEWS, PROJECTION_DIM)


# speedup vs baseline: 1.5289x; 1.5289x over previous
import jax
import jax.numpy as jnp
from jax.experimental import pallas as pl

NUM_VIEWS = 12
PROJECTION_DIM = 512
BATCH = 4096

BLOCK_B = 256


def _body(x_ref, p_ref, o_ref):
    o_ref[...] = x_ref[...] + p_ref[...]


def kernel(onedimage, pos_table):
    out = pl.pallas_call(
        _body,
        grid=(BATCH // BLOCK_B,),
        in_specs=[
            pl.BlockSpec((BLOCK_B, NUM_VIEWS, PROJECTION_DIM), lambda i: (i, 0, 0)),
            pl.BlockSpec((NUM_VIEWS, PROJECTION_DIM), lambda i: (0, 0)),
        ],
        out_specs=pl.BlockSpec((BLOCK_B, NUM_VIEWS, PROJECTION_DIM), lambda i: (i, 0, 0)),
        out_shape=jax.ShapeDtypeStruct((BATCH, NUM_VIEWS, PROJECTION_DIM), jnp.float32),
    )(onedimage, pos_table)
    return out
